# baseline (speedup 1.0000x reference)
import jax
import jax.numpy as jnp
from jax import lax
from jax.experimental import pallas as pl
from jax.experimental.pallas import tpu as pltpu
from jax.experimental.pallas import tpu_sc as plsc

TOTAL = 16384
B = 16
V = 100000
D = 64
H = 64
DP = 128

NC = 2
NS = 16
L = 16
NW = NC * NS
CH = TOTAL // NW
SUB = 128
NSUB = CH // SUB

TBLK = 256
NBLK = TOTAL // TBLK

VBLK = 2000


def _pad_body(t_ref, o_ref):
    o_ref[:, 0:D] = t_ref[...]


def _tc_pad_table(table):
    return pl.pallas_call(
        _pad_body,
        grid=(V // VBLK,),
        in_specs=[pl.BlockSpec((VBLK, D), lambda i: (i, 0))],
        out_specs=pl.BlockSpec((VBLK, DP), lambda i: (i, 0)),
        out_shape=jax.ShapeDtypeStruct((V, DP), jnp.float32),
    )(table)


def _splat(off_vec, j):
    return lax.gather(
        off_vec,
        jnp.full((L, 1), j, jnp.int32),
        lax.GatherDimensionNumbers(
            offset_dims=(), collapsed_slice_dims=(0,), start_index_map=(0,)),
        (1,),
        mode=lax.GatherScatterMode.PROMISE_IN_BOUNDS,
    )


def _sc_body(x_hbm, off_hbm, table_hbm, out_hbm, ids_v, off_v, dst_v, rows_v, sem):
    cid = lax.axis_index("c")
    sid = lax.axis_index("s")
    wid = sid * NC + cid
    base = wid * CH
    pltpu.sync_copy(x_hbm.at[pl.ds(base, CH)], ids_v)
    pltpu.sync_copy(off_hbm, off_v)
    off_vec = off_v[...]
    offs = [_splat(off_vec, j) for j in range(B)]
    for j in range(CH // L):
        pos = jnp.full((L,), base + j * L, jnp.int32) + lax.iota(jnp.int32, L)
        cnt = jnp.zeros((L,), jnp.int32)
        start = jnp.zeros((L,), jnp.int32)
        for ob in offs:
            ge = pos >= ob
            cnt = cnt + jnp.where(ge, 1, 0).astype(jnp.int32)
            start = jnp.maximum(start, jnp.where(ge, ob, 0))
        dst = (pos - start) * B + (cnt - 1)
        dst_v[j // (SUB // L), pl.ds((j % (SUB // L)) * L, L)] = dst
    for s in range(NSUB):
        pltpu.async_copy(
            table_hbm.at[ids_v.at[pl.ds(s * SUB, SUB)]], rows_v, sem
        ).wait()
        pltpu.sync_copy(rows_v, out_hbm.at[dst_v.at[s]])


def _sc_gather_pad(x, offset, table128):
    mesh = plsc.VectorSubcoreMesh(core_axis_name="c", subcore_axis_name="s")
    return pl.kernel(
        _sc_body,
        out_type=jax.ShapeDtypeStruct((TOTAL * B, DP), jnp.float32),
        mesh=mesh,
        scratch_types=[
            pltpu.VMEM((CH,), jnp.int32),
            pltpu.VMEM((B,), jnp.int32),
            pltpu.VMEM((NSUB, SUB), jnp.int32),
            pltpu.VMEM((SUB, DP), jnp.float32),
            pltpu.SemaphoreType.DMA,
        ],
    )(x, offset, table128)


def _tc_gru_body(maxlen_ref, lens_ref, p_ref, wi01_ref, win_ref,
                 wh01_ref, whn_ref, dw_ref, db_ref,
                 out_ref, h_ref, g01_ref, gin_ref):
    i = pl.program_id(0)

    @pl.when(i == 0)
    def _init():
        h_ref[...] = jnp.zeros((B, H), jnp.float32)

    @pl.when(i * TBLK < maxlen_ref[0])
    def _compute():
        lens = lens_ref[...]
        wh01 = wh01_ref[...]
        whn = whn_ref[...]
        tbase = i * TBLK
        blk = p_ref[...][:, :, 0:D]
        dims = (((2,), (0,)), ((), ()))
        g01 = lax.dot_general(blk, wi01_ref[...], dims,
                              preferred_element_type=jnp.float32)
        gin = lax.dot_general(blk, win_ref[...], dims,
                              preferred_element_type=jnp.float32)
        tmask = (lax.broadcasted_iota(jnp.int32, (TBLK, B, 1), 0) + tbase
                 < lens.reshape(1, B, 1))
        col = lax.broadcasted_iota(jnp.int32, (1, 1, 2 * H), 2)
        alt01 = jnp.where(col >= H, 1e9, 0.0).astype(jnp.float32)
        g01_ref[...] = jnp.where(tmask, g01, alt01)
        gin_ref[...] = jnp.where(tmask, gin, 0.0)

        def step(tl, h):
            gh01 = jnp.dot(h, wh01, preferred_element_type=jnp.float32)
            ghn = jnp.dot(h, whn, preferred_element_type=jnp.float32)
            s = jax.nn.sigmoid(g01_ref[tl] + gh01)
            n = jnp.tanh(gin_ref[tl] + s[:, 0:H] * ghn)
            z = s[:, H:2 * H]
            return (1.0 - z) * n + z * h

        h_ref[...] = lax.fori_loop(0, TBLK, step, h_ref[...], unroll=8)

    @pl.when(i == NBLK - 1)
    def _finalize():
        h = h_ref[...]
        o = jnp.dot(h, dw_ref[...], preferred_element_type=jnp.float32) + db_ref[...]
        nrm = jnp.sqrt(jnp.sum(o * o, axis=1, keepdims=True))
        out_ref[...] = o / jnp.maximum(nrm, 1e-12)


def _tc_gru(p, lens, maxlen, wi01, win, wh01, whn, dense_W, dense_b):
    def p_index(i, mref):
        nact = (mref[0] + TBLK - 1) // TBLK
        return (jnp.minimum(i, jnp.maximum(nact - 1, 0)), 0, 0)

    return pl.pallas_call(
        _tc_gru_body,
        grid_spec=pltpu.PrefetchScalarGridSpec(
            num_scalar_prefetch=1,
            grid=(NBLK,),
            in_specs=[
                pl.BlockSpec((B, 1), lambda i, mref: (0, 0)),
                pl.BlockSpec((TBLK, B, DP), p_index),
                pl.BlockSpec((D, 2 * H), lambda i, mref: (0, 0)),
                pl.BlockSpec((D, H), lambda i, mref: (0, 0)),
                pl.BlockSpec((H, 2 * H), lambda i, mref: (0, 0)),
                pl.BlockSpec((H, H), lambda i, mref: (0, 0)),
                pl.BlockSpec((H, D), lambda i, mref: (0, 0)),
                pl.BlockSpec((1, D), lambda i, mref: (0, 0)),
            ],
            out_specs=pl.BlockSpec((B, D), lambda i, mref: (0, 0)),
            scratch_shapes=[
                pltpu.VMEM((B, H), jnp.float32),
                pltpu.VMEM((TBLK, B, 2 * H), jnp.float32),
                pltpu.VMEM((TBLK, B, H), jnp.float32),
            ],
        ),
        out_shape=jax.ShapeDtypeStruct((B, D), jnp.float32),
    )(maxlen, lens, p, wi01, win, wh01, whn, dense_W, dense_b)


def kernel(x, offset, table, W_ih, W_hh, dense_W, dense_b):
    bounds = jnp.concatenate([offset, jnp.full((1,), TOTAL, jnp.int32)])
    lengths = bounds[1:] - bounds[:-1]
    maxlen = jnp.max(lengths).reshape((1,))
    table128 = _tc_pad_table(table)
    p = _sc_gather_pad(x, offset, table128).reshape(TOTAL, B, DP)
    wi01 = jnp.concatenate([W_ih[0:H].T, W_ih[H:2 * H].T], axis=1)
    win = W_ih[2 * H:3 * H].T
    wh01 = jnp.concatenate([W_hh[0:H].T, W_hh[H:2 * H].T], axis=1)
    whn = W_hh[2 * H:3 * H].T
    return _tc_gru(
        p,
        lengths.reshape(B, 1),
        maxlen,
        wi01,
        win,
        wh01,
        whn,
        dense_W,
        dense_b.reshape(1, D),
    )


# speedup vs baseline: 1.7448x; 1.7448x over previous
import jax
import jax.numpy as jnp
from jax import lax
from jax.experimental import pallas as pl
from jax.experimental.pallas import tpu as pltpu
from jax.experimental.pallas import tpu_sc as plsc

TOTAL = 16384
B = 16
V = 100000
D = 64
H = 64
DP = 128

NC = 2
NS = 16
L = 16
NW = NC * NS
CH = TOTAL // NW
SUB = 128
NSUB = CH // SUB

TBLK = 256
NBLK = TOTAL // TBLK

VBLK = 2000


def _pad_body(t_ref, o_ref):
    o_ref[:, 0:D] = t_ref[...]


def _tc_pad_table(table):
    return pl.pallas_call(
        _pad_body,
        grid=(V // VBLK,),
        in_specs=[pl.BlockSpec((VBLK, D), lambda i: (i, 0))],
        out_specs=pl.BlockSpec((VBLK, DP), lambda i: (i, 0)),
        out_shape=jax.ShapeDtypeStruct((V, DP), jnp.float32),
    )(table)


def _splat(off_vec, j):
    return lax.gather(
        off_vec,
        jnp.full((L, 1), j, jnp.int32),
        lax.GatherDimensionNumbers(
            offset_dims=(), collapsed_slice_dims=(0,), start_index_map=(0,)),
        (1,),
        mode=lax.GatherScatterMode.PROMISE_IN_BOUNDS,
    )


def _sc_body(x_hbm, off_hbm, table_hbm, out_hbm, ids_v, off_v, dst_v, rows_v, sem):
    cid = lax.axis_index("c")
    sid = lax.axis_index("s")
    wid = sid * NC + cid
    base = wid * CH
    pltpu.sync_copy(x_hbm.at[pl.ds(base, CH)], ids_v)
    pltpu.sync_copy(off_hbm, off_v)
    off_vec = off_v[...]
    offs = [_splat(off_vec, j) for j in range(B)]
    for j in range(CH // L):
        pos = jnp.full((L,), base + j * L, jnp.int32) + lax.iota(jnp.int32, L)
        cnt = jnp.zeros((L,), jnp.int32)
        start = jnp.zeros((L,), jnp.int32)
        for ob in offs:
            ge = pos >= ob
            cnt = cnt + jnp.where(ge, 1, 0).astype(jnp.int32)
            start = jnp.maximum(start, jnp.where(ge, ob, 0))
        dst = (pos - start) * B + (cnt - 1)
        dst_v[j // (SUB // L), pl.ds((j % (SUB // L)) * L, L)] = dst
    for s in range(NSUB):
        pltpu.async_copy(
            table_hbm.at[ids_v.at[pl.ds(s * SUB, SUB)]], rows_v, sem
        ).wait()
        pltpu.sync_copy(rows_v, out_hbm.at[dst_v.at[s]])


def _sc_gather_pad(x, offset, table128):
    mesh = plsc.VectorSubcoreMesh(core_axis_name="c", subcore_axis_name="s")
    return pl.kernel(
        _sc_body,
        out_type=jax.ShapeDtypeStruct((TOTAL * B, DP), jnp.float32),
        mesh=mesh,
        scratch_types=[
            pltpu.VMEM((CH,), jnp.int32),
            pltpu.VMEM((B,), jnp.int32),
            pltpu.VMEM((NSUB, SUB), jnp.int32),
            pltpu.VMEM((SUB, DP), jnp.float32),
            pltpu.SemaphoreType.DMA,
        ],
    )(x, offset, table128)


def _tc_gru_body(maxlen_ref, lens_ref, p_ref, wir_ref, wiz_ref, win_ref,
                 whr_ref, whz_ref, whn_ref, dw_ref, db_ref,
                 out_ref, h_ref, gir_ref, giz_ref, gin_ref):
    i = pl.program_id(0)

    @pl.when(i == 0)
    def _init():
        h_ref[...] = jnp.zeros((B, H), jnp.float32)

    @pl.when(i * TBLK < maxlen_ref[0])
    def _compute():
        lens = lens_ref[...]
        whr = whr_ref[...]
        whz = whz_ref[...]
        whn = whn_ref[...]
        tbase = i * TBLK
        blk = p_ref[...][:, :, 0:D]
        dims = (((2,), (0,)), ((), ()))
        gir = lax.dot_general(blk, wir_ref[...], dims,
                              preferred_element_type=jnp.float32)
        giz = lax.dot_general(blk, wiz_ref[...], dims,
                              preferred_element_type=jnp.float32)
        gin = lax.dot_general(blk, win_ref[...], dims,
                              preferred_element_type=jnp.float32)
        tmask = (lax.broadcasted_iota(jnp.int32, (TBLK, B, 1), 0) + tbase
                 < lens.reshape(1, B, 1))
        gir_ref[...] = jnp.where(tmask, gir, 0.0)
        giz_ref[...] = jnp.where(tmask, giz, 1e9)
        gin_ref[...] = jnp.where(tmask, gin, 0.0)

        def step(tl, h):
            ghr = jnp.dot(h, whr, preferred_element_type=jnp.float32)
            ghz = jnp.dot(h, whz, preferred_element_type=jnp.float32)
            ghn = jnp.dot(h, whn, preferred_element_type=jnp.float32)
            r = jax.nn.sigmoid(gir_ref[tl] + ghr)
            z = jax.nn.sigmoid(giz_ref[tl] + ghz)
            n = jnp.tanh(gin_ref[tl] + r * ghn)
            return (1.0 - z) * n + z * h

        h_ref[...] = lax.fori_loop(0, TBLK, step, h_ref[...], unroll=8)

    @pl.when(i == NBLK - 1)
    def _finalize():
        h = h_ref[...]
        o = jnp.dot(h, dw_ref[...], preferred_element_type=jnp.float32) + db_ref[...]
        nrm = jnp.sqrt(jnp.sum(o * o, axis=1, keepdims=True))
        out_ref[...] = o / jnp.maximum(nrm, 1e-12)


def _tc_gru(p, lens, maxlen, wih_t, whh_t, dense_W, dense_b):
    def p_index(i, mref):
        nact = (mref[0] + TBLK - 1) // TBLK
        return (jnp.minimum(i, jnp.maximum(nact - 1, 0)), 0, 0)

    return pl.pallas_call(
        _tc_gru_body,
        grid_spec=pltpu.PrefetchScalarGridSpec(
            num_scalar_prefetch=1,
            grid=(NBLK,),
            in_specs=[
                pl.BlockSpec((B, 1), lambda i, mref: (0, 0)),
                pl.BlockSpec((TBLK, B, DP), p_index),
                pl.BlockSpec((D, H), lambda i, mref: (0, 0)),
                pl.BlockSpec((D, H), lambda i, mref: (0, 0)),
                pl.BlockSpec((D, H), lambda i, mref: (0, 0)),
                pl.BlockSpec((H, H), lambda i, mref: (0, 0)),
                pl.BlockSpec((H, H), lambda i, mref: (0, 0)),
                pl.BlockSpec((H, H), lambda i, mref: (0, 0)),
                pl.BlockSpec((H, D), lambda i, mref: (0, 0)),
                pl.BlockSpec((1, D), lambda i, mref: (0, 0)),
            ],
            out_specs=pl.BlockSpec((B, D), lambda i, mref: (0, 0)),
            scratch_shapes=[
                pltpu.VMEM((B, H), jnp.float32),
                pltpu.VMEM((TBLK, B, H), jnp.float32),
                pltpu.VMEM((TBLK, B, H), jnp.float32),
                pltpu.VMEM((TBLK, B, H), jnp.float32),
            ],
        ),
        out_shape=jax.ShapeDtypeStruct((B, D), jnp.float32),
    )(maxlen, lens, p, *wih_t, *whh_t, dense_W, dense_b)


def kernel(x, offset, table, W_ih, W_hh, dense_W, dense_b):
    bounds = jnp.concatenate([offset, jnp.full((1,), TOTAL, jnp.int32)])
    lengths = bounds[1:] - bounds[:-1]
    maxlen = jnp.max(lengths).reshape((1,))
    table128 = _tc_pad_table(table)
    p = _sc_gather_pad(x, offset, table128).reshape(TOTAL, B, DP)
    wih_t = (W_ih[0:H].T, W_ih[H:2 * H].T, W_ih[2 * H:3 * H].T)
    whh_t = (W_hh[0:H].T, W_hh[H:2 * H].T, W_hh[2 * H:3 * H].T)
    return _tc_gru(
        p,
        lengths.reshape(B, 1),
        maxlen,
        wih_t,
        whh_t,
        dense_W,
        dense_b.reshape(1, D),
    )
